# baseline (speedup 1.0000x reference)
import functools

import jax
import jax.numpy as jnp
from jax import lax
from jax.experimental import pallas as pl
from jax.experimental.pallas import tpu as pltpu
from jax.experimental.pallas import tpu_sc as plsc

_MIN_STREAK_HIGH = 4
_MIN_STREAK_LOW = 7
_MIN_STREAKS_HIGH_ACT = 2
_MIN_STREAKS_LOW_ACT = 3
_HIGH_IMPACT_INITIAL = 0.18
_LOW_IMPACT_INITIAL = 0.05
_MAX_ADJUSTMENT = 0.35
_HIGH_DECAY = 0.5
_LOW_DECAY = 0.5

_L = 16
_PAD = 16


def _make_sc_kernel(B, T):
    mesh = plsc.VectorSubcoreMesh(core_axis_name="c", subcore_axis_name="s")
    nsteps = T // _L

    @functools.partial(
        pl.kernel,
        mesh=mesh,
        out_type=jax.ShapeDtypeStruct((B * _L,), jnp.float32),
        scratch_types=[
            pltpu.VMEM((T + 2 * _PAD,), jnp.int32),
            pltpu.VMEM((_L,), jnp.float32),
            pltpu.VMEM((_L,), jnp.float32),
        ],
    )
    def sc_kernel(d_hbm, g_hbm, out_hbm, gpad_v, d_v, res_v):
        c = lax.axis_index("c")
        s = lax.axis_index("s")
        wid = s * 2 + c

        @pl.when(wid < B)
        def _():
            zeros = jnp.zeros((_L,), jnp.int32)
            gpad_v[pl.ds(0, _L)] = zeros
            gpad_v[pl.ds(T + _PAD, _L)] = zeros
            pltpu.sync_copy(g_hbm.at[pl.ds(wid * T, T)], gpad_v.at[pl.ds(_PAD, T)])
            pltpu.sync_copy(d_hbm, d_v)

            one = jnp.ones((_L,), jnp.int32)
            zero = jnp.zeros((_L,), jnp.int32)

            def onv(off):
                return jnp.where(gpad_v[pl.ds(off, _L)] == 2, one, zero)

            def step(j, carry):
                hi_acc, lo_acc = carry
                base = _PAD + j * _L
                start = one - onv(base - 1)
                win = onv(base)
                for k in range(1, _MIN_STREAK_HIGH):
                    win = win * onv(base + k)
                hi_acc = hi_acc + win * start
                for k in range(_MIN_STREAK_HIGH, _MIN_STREAK_LOW):
                    win = win * onv(base + k)
                lo_acc = lo_acc + win * start
                return hi_acc, lo_acc

            zacc = jnp.zeros((_L,), jnp.int32)
            hi_acc, lo_acc = lax.fori_loop(0, nsteps, step, (zacc, zacc))

            lane = lax.iota(jnp.int32, _L)

            def lane_allsum(x):
                for shift in (8, 4, 2, 1):
                    x = x + x.at[lane ^ shift].get(mode="promise_in_bounds")
                return x

            hf = lane_allsum(hi_acc).astype(jnp.float32)
            lf = lane_allsum(lo_acc).astype(jnp.float32)
            ha = _HIGH_IMPACT_INITIAL * jnp.exp(
                -_HIGH_DECAY * (hf - _MIN_STREAKS_HIGH_ACT)
            )
            ha = jnp.where(hf >= _MIN_STREAKS_HIGH_ACT, ha, 0.0)
            la = _LOW_IMPACT_INITIAL * jnp.exp(
                -_LOW_DECAY * (lf - _MIN_STREAKS_LOW_ACT)
            )
            la = jnp.where(lf >= _MIN_STREAKS_LOW_ACT, la, 0.0)
            adj = jnp.minimum(ha + la, _MAX_ADJUSTMENT)

            dsel = lane_allsum(jnp.where(lane == wid, d_v[...], 0.0))
            res_v[...] = jnp.clip(dsel + adj, 0.0, 1.0)
            pltpu.sync_copy(res_v, out_hbm.at[pl.ds(wid * _L, _L)])

    return sc_kernel


def kernel(drowsiness_index, gesture_sequence):
    B, T = gesture_sequence.shape[0], gesture_sequence.shape[1]
    gflat = gesture_sequence.reshape(B * T)
    drows = drowsiness_index.reshape(B)
    out = _make_sc_kernel(B, T)(drows, gflat)
    return out.reshape(B, _L)[:, :1]
